# baseline (speedup 1.0000x reference)
import functools

import numpy as np

import jax
import jax.numpy as jnp
from jax import lax
from jax.experimental import pallas as pl
from jax.experimental.pallas import tpu as pltpu
from jax.experimental.pallas import tpu_sc as plsc

_EPS = 1e-8
_L = 16
_NT = 16
_UNROLL = 4

_FI = [np.arange(16 * k, 16 * k + 16) // 3 for k in range(3)]
_CM = [np.arange(16 * k, 16 * k + 16) % 3 for k in range(3)]


def _sqrt16(v):
    i = plsc.bitcast(v, jnp.int32)
    y = plsc.bitcast(jnp.int32(0x5F3759DF) - (i >> 1), jnp.float32)
    y = y * (1.5 - 0.5 * v * y * y)
    y = y * (1.5 - 0.5 * v * y * y)
    y = y * (1.5 - 0.5 * v * y * y)
    r = v * y
    return jnp.where(v > 0.0, r, 0.0)


def _take(v, idx):
    return v[idx]


@functools.partial(jax.jit, static_argnames=("n", "num_segments"))
def _run(fv_r, seg_r, w_b, *, n, num_segments):
    nt = _NT
    per = fv_r.shape[2]
    groups = per // _L
    last = nt - 1
    vlast = n - last * per
    assert 0 < vlast <= per and (vlast * 3) % 8 == 0
    b_pad = ((num_segments + 1 + 127) // 128) * 128
    nblk = b_pad // 128

    mesh = plsc.VectorSubcoreMesh(
        core_axis_name="c", subcore_axis_name="s", num_cores=1)

    @functools.partial(
        pl.kernel,
        out_type=jax.ShapeDtypeStruct((n * 3,), jnp.float32),
        mesh=mesh,
        compiler_params=pltpu.CompilerParams(needs_layout_passes=False),
        scratch_types=[
            pltpu.VMEM((3, per), jnp.float32),
            pltpu.VMEM((per * 3,), jnp.float32),
            pltpu.VMEM((per,), jnp.int32),
            pltpu.VMEM((b_pad,), jnp.float32),
            pltpu.VMEM((b_pad,), jnp.float32),
            pltpu.VMEM((b_pad,), jnp.float32),
            pltpu.VMEM((128,), jnp.float32),
            pltpu.VMEM((_L,), jnp.float32),
            pltpu.VMEM((nt, 2, 128), jnp.float32),
            pltpu.VMEM_SHARED((nt, 2, b_pad), jnp.float32),
            pltpu.VMEM_SHARED((b_pad,), jnp.float32),
        ],
    )
    def sc_kernel(fv_hbm, seg_hbm, w_hbm, out_hbm,
                  fv_v, out_v, seg_v, sums_v, cnts_v, inv_v, invblk_v, w_v,
                  slab_v, shared, inv_sh):
        sid = lax.axis_index("s")
        with jax.named_scope("dma_in"):
            pltpu.sync_copy(fv_hbm.at[sid], fv_v)
            pltpu.sync_copy(seg_hbm.at[sid], seg_v)
            pltpu.sync_copy(w_hbm, w_v)

        zeros = jnp.zeros((_L,), jnp.float32)
        ones = jnp.ones((_L,), jnp.float32)

        with jax.named_scope("zero"):
            def zero_body(j, _):
                sums_v[pl.ds(j * _L, _L)] = zeros
                cnts_v[pl.ds(j * _L, _L)] = zeros
                return 0
            lax.fori_loop(0, b_pad // _L, zero_body, 0)

        with jax.named_scope("acc"):
            def acc_body(gg, _):
                base = gg * (_UNROLL * _L)
                nrms = []
                segs = []
                for u in range(_UNROLL):
                    o = base + u * _L
                    x = fv_v[0, pl.ds(o, _L)]
                    y = fv_v[1, pl.ds(o, _L)]
                    z = fv_v[2, pl.ds(o, _L)]
                    nrms.append(_sqrt16(x * x + y * y + z * z))
                    segs.append(seg_v[pl.ds(o, _L)])
                for u in range(_UNROLL):
                    plsc.addupdate_scatter(sums_v, [segs[u]], nrms[u])
                    plsc.addupdate_scatter(cnts_v, [segs[u]], ones)
                return 0
            lax.fori_loop(0, groups // _UNROLL, acc_body, 0)

        with jax.named_scope("reduce"):
            pltpu.sync_copy(sums_v, shared.at[sid, 0])
            pltpu.sync_copy(cnts_v, shared.at[sid, 1])
            plsc.subcore_barrier()

            @pl.when(sid < nblk)
            def _():
                pltpu.sync_copy(shared.at[:, :, pl.ds(sid * 128, 128)],
                                slab_v)

                def red_body(j, _):
                    o = j * _L
                    s = zeros
                    c = zeros
                    for t in range(nt):
                        s = s + slab_v[t, 0, pl.ds(o, _L)]
                        c = c + slab_v[t, 1, pl.ds(o, _L)]
                    mean = jnp.maximum(s / jnp.maximum(c, 1.0), _EPS)
                    invblk_v[pl.ds(o, _L)] = w_v[...] / mean
                    return 0
                lax.fori_loop(0, 128 // _L, red_body, 0)
                pltpu.sync_copy(invblk_v, inv_sh.at[pl.ds(sid * 128, 128)])

            plsc.subcore_barrier()
            pltpu.sync_copy(inv_sh, inv_v)

        lane = lax.iota(jnp.int32, _L)
        fi, m0, m1 = [], [], []
        for k in range(3):
            v = lane + (16 * k)
            cm = lax.rem(v, 3)
            fi.append(lax.div(v, 3))
            m0.append(cm == 0)
            m1.append(cm == 1)

        with jax.named_scope("scale"):
            def scale_body(gg, _):
                base = gg * (_UNROLL * _L)
                for u in range(_UNROLL):
                    o = base + u * _L
                    seg = seg_v[pl.ds(o, _L)]
                    iv = plsc.load_gather(inv_v, [seg])
                    x = fv_v[0, pl.ds(o, _L)] * iv
                    y = fv_v[1, pl.ds(o, _L)] * iv
                    z = fv_v[2, pl.ds(o, _L)] * iv
                    for k in range(3):
                        vec = jnp.where(
                            m0[k], _take(x, fi[k]),
                            jnp.where(m1[k], _take(y, fi[k]),
                                      _take(z, fi[k])))
                        out_v[pl.ds(o * 3 + k * _L, _L)] = vec
                return 0
            lax.fori_loop(0, groups // _UNROLL, scale_body, 0)

        with jax.named_scope("dma_out"):
            @pl.when(sid < last)
            def _():
                pltpu.sync_copy(out_v,
                                out_hbm.at[pl.ds(sid * (per * 3), per * 3)])

            @pl.when(sid == last)
            def _():
                pltpu.sync_copy(out_v.at[pl.ds(0, vlast * 3)],
                                out_hbm.at[pl.ds(last * (per * 3),
                                                 vlast * 3)])

    return sc_kernel(fv_r, seg_r, w_b)


def kernel(fv_pos, segment_ids, weight):
    n = fv_pos.shape[0]
    num_segments = 1024
    chunk = _L * _UNROLL
    per = ((n + _NT * chunk - 1) // (_NT * chunk)) * chunk
    n_pad = _NT * per

    seg32 = segment_ids.astype(jnp.int32)
    fv_pad = jnp.concatenate(
        [fv_pos.astype(jnp.float32),
         jnp.zeros((n_pad - n, 3), jnp.float32)], axis=0)
    seg_pad = jnp.concatenate(
        [seg32, jnp.full((n_pad - n,), num_segments, jnp.int32)], axis=0)
    fv_r = fv_pad.reshape(_NT, per, 3).transpose(0, 2, 1)
    seg_r = seg_pad.reshape(_NT, per)
    w_b = jnp.broadcast_to(weight.astype(jnp.float32), (_L,))

    out = _run(fv_r, seg_r, w_b, n=n, num_segments=num_segments)
    return out.reshape(n, 3)


# speedup vs baseline: 2.5790x; 2.5790x over previous
import functools

import numpy as np

import jax
import jax.numpy as jnp
from jax import lax
from jax.experimental import pallas as pl
from jax.experimental.pallas import tpu as pltpu
from jax.experimental.pallas import tpu_sc as plsc

_EPS = 1e-8
_L = 16
_NT = 16
_UNROLL = 4

def _sqrt16(v):
    i = plsc.bitcast(v, jnp.int32)
    y = plsc.bitcast(jnp.int32(0x5F3759DF) - (i >> 1), jnp.float32)
    y = y * (1.5 - 0.5 * v * y * y)
    y = y * (1.5 - 0.5 * v * y * y)
    y = y * (1.5 - 0.5 * v * y * y)
    r = v * y
    return jnp.where(v > 0.0, r, 0.0)


@functools.partial(jax.jit, static_argnames=("n", "num_segments"))
def _run(fv_r, seg_r, w_b, *, n, num_segments):
    nt = _NT
    per = fv_r.shape[2]
    groups = per // _L
    b_pad = ((num_segments + 1 + 127) // 128) * 128
    nblk = b_pad // 128

    mesh = plsc.VectorSubcoreMesh(
        core_axis_name="c", subcore_axis_name="s", num_cores=1)

    @functools.partial(
        pl.kernel,
        out_type=jax.ShapeDtypeStruct((nt, 3, per), jnp.float32),
        mesh=mesh,
        compiler_params=pltpu.CompilerParams(needs_layout_passes=False),
        scratch_types=[
            pltpu.VMEM((3, per), jnp.float32),
            pltpu.VMEM((per,), jnp.int32),
            pltpu.VMEM((b_pad,), jnp.float32),
            pltpu.VMEM((b_pad,), jnp.float32),
            pltpu.VMEM((b_pad,), jnp.float32),
            pltpu.VMEM((128,), jnp.float32),
            pltpu.VMEM((_L,), jnp.float32),
            pltpu.VMEM((nt, 2, 128), jnp.float32),
            pltpu.VMEM_SHARED((nt, 2, b_pad), jnp.float32),
            pltpu.VMEM_SHARED((b_pad,), jnp.float32),
        ],
    )
    def sc_kernel(fv_hbm, seg_hbm, w_hbm, out_hbm,
                  fv_v, seg_v, sums_v, cnts_v, inv_v, invblk_v, w_v,
                  slab_v, shared, inv_sh):
        sid = lax.axis_index("s")
        with jax.named_scope("dma_in"):
            pltpu.sync_copy(fv_hbm.at[sid], fv_v)
            pltpu.sync_copy(seg_hbm.at[sid], seg_v)
            pltpu.sync_copy(w_hbm, w_v)

        zeros = jnp.zeros((_L,), jnp.float32)
        ones = jnp.ones((_L,), jnp.float32)

        with jax.named_scope("zero"):
            def zero_body(j, _):
                sums_v[pl.ds(j * _L, _L)] = zeros
                cnts_v[pl.ds(j * _L, _L)] = zeros
                return 0
            lax.fori_loop(0, b_pad // _L, zero_body, 0)

        with jax.named_scope("acc"):
            def acc_body(gg, _):
                base = gg * (_UNROLL * _L)
                nrms = []
                segs = []
                for u in range(_UNROLL):
                    o = base + u * _L
                    x = fv_v[0, pl.ds(o, _L)]
                    y = fv_v[1, pl.ds(o, _L)]
                    z = fv_v[2, pl.ds(o, _L)]
                    nrms.append(_sqrt16(x * x + y * y + z * z))
                    segs.append(seg_v[pl.ds(o, _L)])
                for u in range(_UNROLL):
                    plsc.addupdate_scatter(sums_v, [segs[u]], nrms[u])
                    plsc.addupdate_scatter(cnts_v, [segs[u]], ones)
                return 0
            lax.fori_loop(0, groups // _UNROLL, acc_body, 0)

        with jax.named_scope("reduce"):
            pltpu.sync_copy(sums_v, shared.at[sid, 0])
            pltpu.sync_copy(cnts_v, shared.at[sid, 1])
            plsc.subcore_barrier()

            @pl.when(sid < nblk)
            def _():
                pltpu.sync_copy(shared.at[:, :, pl.ds(sid * 128, 128)],
                                slab_v)

                def red_body(j, _):
                    o = j * _L
                    s = zeros
                    c = zeros
                    for t in range(nt):
                        s = s + slab_v[t, 0, pl.ds(o, _L)]
                        c = c + slab_v[t, 1, pl.ds(o, _L)]
                    mean = jnp.maximum(s / jnp.maximum(c, 1.0), _EPS)
                    invblk_v[pl.ds(o, _L)] = w_v[...] / mean
                    return 0
                lax.fori_loop(0, 128 // _L, red_body, 0)
                pltpu.sync_copy(invblk_v, inv_sh.at[pl.ds(sid * 128, 128)])

            plsc.subcore_barrier()
            pltpu.sync_copy(inv_sh, inv_v)

        with jax.named_scope("scale"):
            def scale_body(gg, _):
                base = gg * (_UNROLL * _L)
                ivs = []
                for u in range(_UNROLL):
                    o = base + u * _L
                    seg = seg_v[pl.ds(o, _L)]
                    ivs.append(plsc.load_gather(inv_v, [seg]))
                for u in range(_UNROLL):
                    o = base + u * _L
                    iv = ivs[u]
                    fv_v[0, pl.ds(o, _L)] = fv_v[0, pl.ds(o, _L)] * iv
                    fv_v[1, pl.ds(o, _L)] = fv_v[1, pl.ds(o, _L)] * iv
                    fv_v[2, pl.ds(o, _L)] = fv_v[2, pl.ds(o, _L)] * iv
                return 0
            lax.fori_loop(0, groups // _UNROLL, scale_body, 0)

        with jax.named_scope("dma_out"):
            pltpu.sync_copy(fv_v, out_hbm.at[sid])

    return sc_kernel(fv_r, seg_r, w_b)


def kernel(fv_pos, segment_ids, weight):
    n = fv_pos.shape[0]
    num_segments = 1024
    chunk = _L * _UNROLL
    per = ((n + _NT * chunk - 1) // (_NT * chunk)) * chunk
    n_pad = _NT * per

    seg32 = segment_ids.astype(jnp.int32)
    fv_pad = jnp.concatenate(
        [fv_pos.astype(jnp.float32),
         jnp.zeros((n_pad - n, 3), jnp.float32)], axis=0)
    seg_pad = jnp.concatenate(
        [seg32, jnp.full((n_pad - n,), num_segments, jnp.int32)], axis=0)
    fv_r = fv_pad.reshape(_NT, per, 3).transpose(0, 2, 1)
    seg_r = seg_pad.reshape(_NT, per)
    w_b = jnp.broadcast_to(weight.astype(jnp.float32), (_L,))

    out = _run(fv_r, seg_r, w_b, n=n, num_segments=num_segments)
    return out.transpose(0, 2, 1).reshape(n_pad, 3)[:n]


# speedup vs baseline: 2.6037x; 1.0096x over previous
import functools

import jax
import jax.numpy as jnp
from jax import lax
from jax.experimental import pallas as pl
from jax.experimental.pallas import tpu as pltpu
from jax.experimental.pallas import tpu_sc as plsc

_EPS = 1e-8
_L = 16
_NT = 16
_UNROLL = 4

def _sqrt16(v):
    i = plsc.bitcast(v, jnp.int32)
    y = plsc.bitcast(jnp.int32(0x5F3759DF) - (i >> 1), jnp.float32)
    y = y * (1.5 - 0.5 * v * y * y)
    y = y * (1.5 - 0.5 * v * y * y)
    r = v * y
    return jnp.where(v > 0.0, r, 0.0)


@functools.partial(jax.jit, static_argnames=("n", "num_segments"))
def _run(fv_r, seg_r, w_b, *, n, num_segments):
    nt = _NT
    per = fv_r.shape[2]
    groups = per // _L
    b_pad = ((num_segments + 1 + 127) // 128) * 128
    nblk = b_pad // 128

    mesh = plsc.VectorSubcoreMesh(
        core_axis_name="c", subcore_axis_name="s", num_cores=1)

    @functools.partial(
        pl.kernel,
        out_type=jax.ShapeDtypeStruct((nt, 3, per), jnp.float32),
        mesh=mesh,
        compiler_params=pltpu.CompilerParams(needs_layout_passes=False),
        scratch_types=[
            pltpu.VMEM((3, per), jnp.float32),
            pltpu.VMEM((per,), jnp.int32),
            pltpu.VMEM((b_pad,), jnp.float32),
            pltpu.VMEM((b_pad,), jnp.float32),
            pltpu.VMEM((b_pad,), jnp.float32),
            pltpu.VMEM((128,), jnp.float32),
            pltpu.VMEM((_L,), jnp.float32),
            pltpu.VMEM((nt, 2, 128), jnp.float32),
            pltpu.VMEM_SHARED((nt, 2, b_pad), jnp.float32),
            pltpu.VMEM_SHARED((b_pad,), jnp.float32),
        ],
    )
    def sc_kernel(fv_hbm, seg_hbm, w_hbm, out_hbm,
                  fv_v, seg_v, sums_v, cnts_v, inv_v, invblk_v, w_v,
                  slab_v, shared, inv_sh):
        sid = lax.axis_index("s")
        with jax.named_scope("dma_in"):
            pltpu.sync_copy(fv_hbm.at[sid], fv_v)
            pltpu.sync_copy(seg_hbm.at[sid], seg_v)
            pltpu.sync_copy(w_hbm, w_v)

        zeros = jnp.zeros((_L,), jnp.float32)
        ones = jnp.ones((_L,), jnp.float32)

        with jax.named_scope("zero"):
            def zero_body(j, _):
                sums_v[pl.ds(j * _L, _L)] = zeros
                cnts_v[pl.ds(j * _L, _L)] = zeros
                return 0
            lax.fori_loop(0, b_pad // _L, zero_body, 0)

        with jax.named_scope("acc"):
            def acc_body(gg, _):
                base = gg * (_UNROLL * _L)
                nrms = []
                segs = []
                for u in range(_UNROLL):
                    o = base + u * _L
                    x = fv_v[0, pl.ds(o, _L)]
                    y = fv_v[1, pl.ds(o, _L)]
                    z = fv_v[2, pl.ds(o, _L)]
                    nrms.append(_sqrt16(x * x + y * y + z * z))
                    segs.append(seg_v[pl.ds(o, _L)])
                for u in range(_UNROLL):
                    plsc.addupdate_scatter(sums_v, [segs[u]], nrms[u])
                    plsc.addupdate_scatter(cnts_v, [segs[u]], ones)
                return 0
            lax.fori_loop(0, groups // _UNROLL, acc_body, 0)

        with jax.named_scope("reduce"):
            pltpu.sync_copy(sums_v, shared.at[sid, 0])
            pltpu.sync_copy(cnts_v, shared.at[sid, 1])
            plsc.subcore_barrier()

            @pl.when(sid < nblk)
            def _():
                pltpu.sync_copy(shared.at[:, :, pl.ds(sid * 128, 128)],
                                slab_v)

                def red_body(j, _):
                    o = j * _L
                    s = zeros
                    c = zeros
                    for t in range(nt):
                        s = s + slab_v[t, 0, pl.ds(o, _L)]
                        c = c + slab_v[t, 1, pl.ds(o, _L)]
                    mean = jnp.maximum(s / jnp.maximum(c, 1.0), _EPS)
                    invblk_v[pl.ds(o, _L)] = w_v[...] / mean
                    return 0
                lax.fori_loop(0, 128 // _L, red_body, 0)
                pltpu.sync_copy(invblk_v, inv_sh.at[pl.ds(sid * 128, 128)])

            plsc.subcore_barrier()
            pltpu.sync_copy(inv_sh, inv_v)

        with jax.named_scope("scale"):
            def scale_body(gg, _):
                base = gg * (_UNROLL * _L)
                ivs = []
                for u in range(_UNROLL):
                    o = base + u * _L
                    seg = seg_v[pl.ds(o, _L)]
                    ivs.append(plsc.load_gather(inv_v, [seg]))
                for u in range(_UNROLL):
                    o = base + u * _L
                    iv = ivs[u]
                    fv_v[0, pl.ds(o, _L)] = fv_v[0, pl.ds(o, _L)] * iv
                    fv_v[1, pl.ds(o, _L)] = fv_v[1, pl.ds(o, _L)] * iv
                    fv_v[2, pl.ds(o, _L)] = fv_v[2, pl.ds(o, _L)] * iv
                return 0
            lax.fori_loop(0, groups // _UNROLL, scale_body, 0)

        with jax.named_scope("dma_out"):
            pltpu.sync_copy(fv_v, out_hbm.at[sid])

    return sc_kernel(fv_r, seg_r, w_b)


def kernel(fv_pos, segment_ids, weight):
    n = fv_pos.shape[0]
    num_segments = 1024
    chunk = _L * _UNROLL
    per = ((n + _NT * chunk - 1) // (_NT * chunk)) * chunk
    n_pad = _NT * per

    seg32 = segment_ids.astype(jnp.int32)
    fv_pad = jnp.concatenate(
        [fv_pos.astype(jnp.float32),
         jnp.zeros((n_pad - n, 3), jnp.float32)], axis=0)
    seg_pad = jnp.concatenate(
        [seg32, jnp.full((n_pad - n,), num_segments, jnp.int32)], axis=0)
    fv_r = fv_pad.reshape(_NT, per, 3).transpose(0, 2, 1)
    seg_r = seg_pad.reshape(_NT, per)
    w_b = jnp.broadcast_to(weight.astype(jnp.float32), (_L,))

    out = _run(fv_r, seg_r, w_b, n=n, num_segments=num_segments)
    return out.transpose(0, 2, 1).reshape(n_pad, 3)[:n]


# speedup vs baseline: 2.8103x; 1.0793x over previous
import functools

import jax
import jax.numpy as jnp
from jax import lax
from jax.experimental import pallas as pl
from jax.experimental.pallas import tpu as pltpu
from jax.experimental.pallas import tpu_sc as plsc

_EPS = 1e-8
_L = 16
_NT = 16
_UNROLL = 8

def _sqrt16(v):
    i = plsc.bitcast(v, jnp.int32)
    y = plsc.bitcast(jnp.int32(0x5F3759DF) - (i >> 1), jnp.float32)
    y = y * (1.5 - 0.5 * v * y * y)
    y = y * (1.5 - 0.5 * v * y * y)
    r = v * y
    return jnp.where(v > 0.0, r, 0.0)


@functools.partial(jax.jit, static_argnames=("n", "num_segments"))
def _run(fv_r, seg_r, w_b, *, n, num_segments):
    nt = _NT
    per = fv_r.shape[2]
    groups = per // _L
    b_pad = ((num_segments + 1 + 127) // 128) * 128
    nblk = b_pad // 128

    mesh = plsc.VectorSubcoreMesh(
        core_axis_name="c", subcore_axis_name="s", num_cores=1)

    @functools.partial(
        pl.kernel,
        out_type=jax.ShapeDtypeStruct((nt, 3, per), jnp.float32),
        mesh=mesh,
        compiler_params=pltpu.CompilerParams(needs_layout_passes=False),
        scratch_types=[
            pltpu.VMEM((3, per), jnp.float32),
            pltpu.VMEM((per,), jnp.int32),
            pltpu.VMEM((b_pad,), jnp.float32),
            pltpu.VMEM((b_pad,), jnp.float32),
            pltpu.VMEM((b_pad,), jnp.float32),
            pltpu.VMEM((128,), jnp.float32),
            pltpu.VMEM((_L,), jnp.float32),
            pltpu.VMEM((nt, 2, 128), jnp.float32),
            pltpu.VMEM_SHARED((nt, 2, b_pad), jnp.float32),
            pltpu.VMEM_SHARED((b_pad,), jnp.float32),
        ],
    )
    def sc_kernel(fv_hbm, seg_hbm, w_hbm, out_hbm,
                  fv_v, seg_v, sums_v, cnts_v, inv_v, invblk_v, w_v,
                  slab_v, shared, inv_sh):
        sid = lax.axis_index("s")
        with jax.named_scope("dma_in"):
            pltpu.sync_copy(fv_hbm.at[sid], fv_v)
            pltpu.sync_copy(seg_hbm.at[sid], seg_v)
            pltpu.sync_copy(w_hbm, w_v)

        zeros = jnp.zeros((_L,), jnp.float32)
        ones = jnp.ones((_L,), jnp.float32)

        with jax.named_scope("zero"):
            def zero_body(j, _):
                sums_v[pl.ds(j * _L, _L)] = zeros
                cnts_v[pl.ds(j * _L, _L)] = zeros
                return 0
            lax.fori_loop(0, b_pad // _L, zero_body, 0)

        with jax.named_scope("acc"):
            def acc_body(gg, _):
                base = gg * (_UNROLL * _L)
                nrms = []
                segs = []
                for u in range(_UNROLL):
                    o = base + u * _L
                    x = fv_v[0, pl.ds(o, _L)]
                    y = fv_v[1, pl.ds(o, _L)]
                    z = fv_v[2, pl.ds(o, _L)]
                    nrms.append(_sqrt16(x * x + y * y + z * z))
                    segs.append(seg_v[pl.ds(o, _L)])
                for u in range(_UNROLL):
                    plsc.addupdate_scatter(sums_v, [segs[u]], nrms[u])
                    plsc.addupdate_scatter(cnts_v, [segs[u]], ones)
                return 0
            lax.fori_loop(0, groups // _UNROLL, acc_body, 0)

        with jax.named_scope("reduce"):
            pltpu.sync_copy(sums_v, shared.at[sid, 0])
            pltpu.sync_copy(cnts_v, shared.at[sid, 1])
            plsc.subcore_barrier()

            @pl.when(sid < nblk)
            def _():
                pltpu.sync_copy(shared.at[:, :, pl.ds(sid * 128, 128)],
                                slab_v)

                def red_body(j, _):
                    o = j * _L
                    s = zeros
                    c = zeros
                    for t in range(nt):
                        s = s + slab_v[t, 0, pl.ds(o, _L)]
                        c = c + slab_v[t, 1, pl.ds(o, _L)]
                    mean = jnp.maximum(s / jnp.maximum(c, 1.0), _EPS)
                    invblk_v[pl.ds(o, _L)] = w_v[...] / mean
                    return 0
                lax.fori_loop(0, 128 // _L, red_body, 0)
                pltpu.sync_copy(invblk_v, inv_sh.at[pl.ds(sid * 128, 128)])

            plsc.subcore_barrier()
            pltpu.sync_copy(inv_sh, inv_v)

        with jax.named_scope("scale"):
            def scale_body(gg, _):
                base = gg * (_UNROLL * _L)
                ivs = []
                for u in range(_UNROLL):
                    o = base + u * _L
                    seg = seg_v[pl.ds(o, _L)]
                    ivs.append(plsc.load_gather(inv_v, [seg]))
                for u in range(_UNROLL):
                    o = base + u * _L
                    iv = ivs[u]
                    fv_v[0, pl.ds(o, _L)] = fv_v[0, pl.ds(o, _L)] * iv
                    fv_v[1, pl.ds(o, _L)] = fv_v[1, pl.ds(o, _L)] * iv
                    fv_v[2, pl.ds(o, _L)] = fv_v[2, pl.ds(o, _L)] * iv
                return 0
            lax.fori_loop(0, groups // _UNROLL, scale_body, 0)

        with jax.named_scope("dma_out"):
            pltpu.sync_copy(fv_v, out_hbm.at[sid])

    return sc_kernel(fv_r, seg_r, w_b)


def kernel(fv_pos, segment_ids, weight):
    n = fv_pos.shape[0]
    num_segments = 1024
    chunk = _L * _UNROLL
    per = ((n + _NT * chunk - 1) // (_NT * chunk)) * chunk
    n_pad = _NT * per

    seg32 = segment_ids.astype(jnp.int32)
    fv_pad = jnp.concatenate(
        [fv_pos.astype(jnp.float32),
         jnp.zeros((n_pad - n, 3), jnp.float32)], axis=0)
    seg_pad = jnp.concatenate(
        [seg32, jnp.full((n_pad - n,), num_segments, jnp.int32)], axis=0)
    fv_r = fv_pad.reshape(_NT, per, 3).transpose(0, 2, 1)
    seg_r = seg_pad.reshape(_NT, per)
    w_b = jnp.broadcast_to(weight.astype(jnp.float32), (_L,))

    out = _run(fv_r, seg_r, w_b, n=n, num_segments=num_segments)
    return out.transpose(0, 2, 1).reshape(n_pad, 3)[:n]
